# Initial kernel scaffold; baseline (speedup 1.0000x reference)
import jax
import jax.numpy as jnp
from jax.experimental import pallas as pl


def kernel(hist_seq, hist_answers, new_seq, node_emb, corr_emb, pos_emb, W0, b0, W1, W2, W3, Wq, bq, Wk, bk, Wv, bv, Wo, bo, F0w, F0b, F1w, F1b, ln2g, ln2b, ln3g, ln3b, W4, b4):
    raise NotImplementedError("write your pallas kernel here")


# speedup vs baseline: 2.4129x; 2.4129x over previous
import math

import jax
import jax.numpy as jnp
from jax import lax
from jax.experimental import pallas as pl
from jax.experimental.pallas import tpu as pltpu
from jax.experimental.pallas import tpu_sc as plsc

B, L, D, H, V = 1024, 200, 256, 8, 100000
DH = D // H
NW = 32
CHUNK = 256


def _gather_rows_sc(table, idx):
    n = idx.shape[0]
    per_w = n // NW
    iters = per_w // CHUNK
    mesh = plsc.VectorSubcoreMesh(core_axis_name="c", subcore_axis_name="s")

    def body(table_ref, idx_ref, out_ref, idx_v, rows_v, sem):
        wid = lax.axis_index("s") * 2 + lax.axis_index("c")
        base = wid * per_w

        def step(i, carry):
            off = base + i * CHUNK
            pltpu.sync_copy(idx_ref.at[pl.ds(off, CHUNK)], idx_v)
            pltpu.async_copy(table_ref.at[idx_v], rows_v, sem).wait()
            pltpu.sync_copy(rows_v, out_ref.at[pl.ds(off, CHUNK)])
            return carry

        lax.fori_loop(0, iters, step, 0)

    fn = pl.kernel(
        body,
        out_type=jax.ShapeDtypeStruct((n, D), jnp.float32),
        mesh=mesh,
        scratch_types=[
            pltpu.VMEM((CHUNK,), jnp.int32),
            pltpu.VMEM((CHUNK, D), jnp.float32),
            pltpu.SemaphoreType.DMA,
        ],
    )
    return fn(table, idx)


def _ln_rows(x, g, b):
    m = jnp.mean(x, axis=-1, keepdims=True)
    v = jnp.mean((x - m) ** 2, axis=-1, keepdims=True)
    return (x - m) / jnp.sqrt(v + 1e-5) * g + b


def _transformer_body(he_ref, ne_ref, af_ref, P_ref, cd_ref, W0a_ref,
                      Wkf_ref, Wvf_ref, W3_ref, Wq_ref, bq_ref, bk_ref,
                      bv_ref, Wo_ref, bo_ref, F0w_ref, F1w_ref, F0b_ref,
                      F1b_ref, g2_ref, b2_ref, g3_ref, b3_ref, W4_ref,
                      b4_ref, out_ref):
    he = he_ref[0]
    ne = ne_ref[0]
    af = af_ref[0]

    inter = jnp.dot(he, W0a_ref[...]) + P_ref[...] + af * cd_ref[...]
    k = jnp.dot(inter, Wkf_ref[...]) + bk_ref[...]
    v = jnp.dot(inter, Wvf_ref[...]) + bv_ref[...]
    query = jnp.dot(ne, W3_ref[...])
    q = jnp.dot(query, Wq_ref[...]) + bq_ref[...]

    scale = jnp.float32(1.0 / math.sqrt(DH))
    rows = lax.broadcasted_iota(jnp.int32, (L, L), 0)
    cols = lax.broadcasted_iota(jnp.int32, (L, L), 1)
    causal = cols > rows

    ctxs = []
    for h in range(H):
        sl = slice(h * DH, (h + 1) * DH)
        qh, kh, vh = q[:, sl], k[:, sl], v[:, sl]
        s = lax.dot_general(qh, kh, (((1,), (1,)), ((), ())),
                            preferred_element_type=jnp.float32) * scale
        s = jnp.where(causal, jnp.float32(-1e30), s)
        m = jnp.max(s, axis=1, keepdims=True)
        e = jnp.exp(s - m)
        p = e / jnp.sum(e, axis=1, keepdims=True)
        ctxs.append(jnp.dot(p, vh))
    ctx = jnp.concatenate(ctxs, axis=1)

    atn = jnp.dot(ctx, Wo_ref[...]) + bo_ref[...] + query
    atn = _ln_rows(atn, g2_ref[...], b2_ref[...])
    hdn = jnp.maximum(jnp.dot(atn, F0w_ref[...]) + F0b_ref[...], 0.0)
    ffn = jnp.dot(hdn, F1w_ref[...]) + F1b_ref[...]
    ffn = _ln_rows(ffn + atn, g3_ref[...], b3_ref[...])
    out_ref[0] = jnp.dot(ffn, W4_ref[...]) + b4_ref[...]


def _transformer_tc(he, ne, ansf, P, cdelta, W0a, Wkf, Wvf, W3, Wq, bq, bk,
                    bv, Wo, bo, F0w, F1w, F0b, F1b, g2, b2, g3, b3, W4, b4):
    def blk(shape, imap):
        return pl.BlockSpec(shape, imap)

    row = lambda i: (i, 0, 0)
    const2 = lambda i: (0, 0)
    in_specs = [
        blk((1, L, D), row),
        blk((1, L, D), row),
        blk((1, L, 1), row),
        blk((L, D), const2),
        blk((1, D), const2),
        blk((D, D), const2),
        blk((D, D), const2),
        blk((D, D), const2),
        blk((D, D), const2),
        blk((D, D), const2),
        blk((1, D), const2),
        blk((1, D), const2),
        blk((1, D), const2),
        blk((D, D), const2),
        blk((1, D), const2),
        blk((D, D), const2),
        blk((D, D), const2),
        blk((1, D), const2),
        blk((1, D), const2),
        blk((1, D), const2),
        blk((1, D), const2),
        blk((1, D), const2),
        blk((1, D), const2),
        blk((D, 1), const2),
        blk((1, 1), const2),
    ]
    out = pl.pallas_call(
        _transformer_body,
        grid=(B,),
        in_specs=in_specs,
        out_specs=pl.BlockSpec((1, L, 1), row),
        out_shape=jax.ShapeDtypeStruct((B, L, 1), jnp.float32),
        compiler_params=pltpu.CompilerParams(
            dimension_semantics=("arbitrary",)),
    )(he, ne, ansf, P, cdelta, W0a, Wkf, Wvf, W3, Wq, bq, bk, bv, Wo, bo,
      F0w, F1w, F0b, F1b, g2, b2, g3, b3, W4, b4)
    return out


def kernel(hist_seq, hist_answers, new_seq, node_emb, corr_emb, pos_emb, W0,
           b0, W1, W2, W3, Wq, bq, Wk, bk, Wv, bv, Wo, bo, F0w, F0b, F1w,
           F1b, ln2g, ln2b, ln3g, ln3b, W4, b4):
    idx_all = jnp.concatenate(
        [hist_seq.reshape(-1), new_seq.reshape(-1)]).astype(jnp.int32)
    gathered = _gather_rows_sc(node_emb, idx_all)
    he = gathered[: B * L].reshape(B, L, D)
    ne = gathered[B * L:].reshape(B, L, D)

    corrW = corr_emb @ W0[D:]
    P = pos_emb + b0[None, :] + corrW[0][None, :]
    cdelta = (corrW[1] - corrW[0]).reshape(1, D)
    W0a = W0[:D]
    Wkf = W2 @ Wk
    Wvf = W1 @ Wv
    ansf = hist_answers.astype(jnp.float32).reshape(B, L, 1)

    out3 = _transformer_tc(
        he, ne, ansf, P, cdelta, W0a, Wkf, Wvf, W3, Wq,
        bq.reshape(1, D), bk.reshape(1, D), bv.reshape(1, D), Wo,
        bo.reshape(1, D), F0w, F1w, F0b.reshape(1, D), F1b.reshape(1, D),
        ln2g.reshape(1, D), ln2b.reshape(1, D), ln3g.reshape(1, D),
        ln3b.reshape(1, D), W4, b4.reshape(1, 1))
    return out3.reshape(B, L)


# speedup vs baseline: 2.6293x; 1.0897x over previous
import math

import jax
import jax.numpy as jnp
from jax import lax
from jax.experimental import pallas as pl
from jax.experimental.pallas import tpu as pltpu
from jax.experimental.pallas import tpu_sc as plsc

B, L, D, H, V = 1024, 200, 256, 8, 100000
DH = D // H
NW = 32
CHUNK = 256


def _gather_rows_sc(table, idx):
    n = idx.shape[0]
    per_w = n // NW
    iters = per_w // CHUNK
    mesh = plsc.VectorSubcoreMesh(core_axis_name="c", subcore_axis_name="s")

    def body(table_ref, idx_ref, out_ref, idx_v, rows_v, sem):
        wid = lax.axis_index("s") * 2 + lax.axis_index("c")
        base = wid * per_w

        def step(i, carry):
            off = base + i * CHUNK
            pltpu.sync_copy(idx_ref.at[pl.ds(off, CHUNK)], idx_v)
            pltpu.async_copy(table_ref.at[idx_v], rows_v, sem).wait()
            pltpu.sync_copy(rows_v, out_ref.at[pl.ds(off, CHUNK)])
            return carry

        lax.fori_loop(0, iters, step, 0)

    fn = pl.kernel(
        body,
        out_type=jax.ShapeDtypeStruct((n, D), jnp.float32),
        mesh=mesh,
        scratch_types=[
            pltpu.VMEM((CHUNK,), jnp.int32),
            pltpu.VMEM((CHUNK, D), jnp.float32),
            pltpu.SemaphoreType.DMA,
        ],
    )
    return fn(table, idx)


def _ln_rows(x, g, b):
    m = jnp.mean(x, axis=-1, keepdims=True)
    v = jnp.mean((x - m) ** 2, axis=-1, keepdims=True)
    return (x - m) / jnp.sqrt(v + 1e-5) * g + b


def _transformer_body(he_ref, ne_ref, af_ref, P_ref, cd_ref, W0a_ref,
                      Wkf_ref, Wvf_ref, W3_ref, Wq_ref, bq_ref, bk_ref,
                      bv_ref, Wo_ref, bo_ref, F0w_ref, F1w_ref, F0b_ref,
                      F1b_ref, g2_ref, b2_ref, g3_ref, b3_ref, W4_ref,
                      b4_ref, out_ref):
    bf = jnp.bfloat16
    f32 = jnp.float32

    def mm(a, b):
        return jnp.dot(a.astype(bf), b, preferred_element_type=f32)

    he = he_ref[0]
    ne = ne_ref[0]
    af = af_ref[0]

    inter = mm(he, W0a_ref[...]) + P_ref[...] + af * cd_ref[...]
    k = mm(inter, Wkf_ref[...]) + bk_ref[...]
    v = mm(inter, Wvf_ref[...]) + bv_ref[...]
    query = mm(ne, W3_ref[...])
    q = mm(query, Wq_ref[...]) + bq_ref[...]

    scale = jnp.float32(1.0 / math.sqrt(DH))
    rows = lax.broadcasted_iota(jnp.int32, (L, L), 0)
    cols = lax.broadcasted_iota(jnp.int32, (L, L), 1)
    causal = cols > rows

    q16, k16, v16 = q.astype(bf), k.astype(bf), v.astype(bf)
    ctxs = []
    for h in range(H):
        sl = slice(h * DH, (h + 1) * DH)
        qh, kh, vh = q16[:, sl], k16[:, sl], v16[:, sl]
        s = lax.dot_general(qh, kh, (((1,), (1,)), ((), ())),
                            preferred_element_type=f32) * scale
        s = jnp.where(causal, jnp.float32(-1e30), s)
        m = jnp.max(s, axis=1, keepdims=True)
        e = jnp.exp(s - m)
        p = e / jnp.sum(e, axis=1, keepdims=True)
        ctxs.append(jnp.dot(p.astype(bf), vh, preferred_element_type=f32))
    ctx = jnp.concatenate(ctxs, axis=1)

    atn = mm(ctx, Wo_ref[...]) + bo_ref[...] + query
    atn = _ln_rows(atn, g2_ref[...], b2_ref[...])
    hdn = jnp.maximum(mm(atn, F0w_ref[...]) + F0b_ref[...], 0.0)
    ffn = mm(hdn, F1w_ref[...]) + F1b_ref[...]
    ffn = _ln_rows(ffn + atn, g3_ref[...], b3_ref[...])
    out_ref[0] = jnp.dot(ffn, W4_ref[...]) + b4_ref[...]


def _transformer_tc(he, ne, ansf, P, cdelta, W0a, Wkf, Wvf, W3, Wq, bq, bk,
                    bv, Wo, bo, F0w, F1w, F0b, F1b, g2, b2, g3, b3, W4, b4):
    def blk(shape, imap):
        return pl.BlockSpec(shape, imap)

    row = lambda i: (i, 0, 0)
    const2 = lambda i: (0, 0)
    in_specs = [
        blk((1, L, D), row),
        blk((1, L, D), row),
        blk((1, L, 1), row),
        blk((L, D), const2),
        blk((1, D), const2),
        blk((D, D), const2),
        blk((D, D), const2),
        blk((D, D), const2),
        blk((D, D), const2),
        blk((D, D), const2),
        blk((1, D), const2),
        blk((1, D), const2),
        blk((1, D), const2),
        blk((D, D), const2),
        blk((1, D), const2),
        blk((D, D), const2),
        blk((D, D), const2),
        blk((1, D), const2),
        blk((1, D), const2),
        blk((1, D), const2),
        blk((1, D), const2),
        blk((1, D), const2),
        blk((1, D), const2),
        blk((D, 1), const2),
        blk((1, 1), const2),
    ]
    out = pl.pallas_call(
        _transformer_body,
        grid=(B,),
        in_specs=in_specs,
        out_specs=pl.BlockSpec((1, L, 1), row),
        out_shape=jax.ShapeDtypeStruct((B, L, 1), jnp.float32),
        compiler_params=pltpu.CompilerParams(
            dimension_semantics=("arbitrary",)),
    )(he, ne, ansf, P, cdelta, W0a, Wkf, Wvf, W3, Wq, bq, bk, bv, Wo, bo,
      F0w, F1w, F0b, F1b, g2, b2, g3, b3, W4, b4)
    return out


def kernel(hist_seq, hist_answers, new_seq, node_emb, corr_emb, pos_emb, W0,
           b0, W1, W2, W3, Wq, bq, Wk, bk, Wv, bv, Wo, bo, F0w, F0b, F1w,
           F1b, ln2g, ln2b, ln3g, ln3b, W4, b4):
    idx_all = jnp.concatenate(
        [hist_seq.reshape(-1), new_seq.reshape(-1)]).astype(jnp.int32)
    gathered = _gather_rows_sc(node_emb, idx_all)
    he = gathered[: B * L].reshape(B, L, D)
    ne = gathered[B * L:].reshape(B, L, D)

    corrW = corr_emb @ W0[D:]
    P = pos_emb + b0[None, :] + corrW[0][None, :]
    cdelta = (corrW[1] - corrW[0]).reshape(1, D)
    bf = jnp.bfloat16
    W0a = W0[:D].astype(bf)
    Wkf = (W2 @ Wk).astype(bf)
    Wvf = (W1 @ Wv).astype(bf)
    W3b = W3.astype(bf)
    Wqb = Wq.astype(bf)
    Wob = Wo.astype(bf)
    F0wb = F0w.astype(bf)
    F1wb = F1w.astype(bf)
    ansf = hist_answers.astype(jnp.float32).reshape(B, L, 1)

    out3 = _transformer_tc(
        he, ne, ansf, P, cdelta, W0a, Wkf, Wvf, W3b, Wqb,
        bq.reshape(1, D), bk.reshape(1, D), bv.reshape(1, D), Wob,
        bo.reshape(1, D), F0wb, F1wb, F0b.reshape(1, D), F1b.reshape(1, D),
        ln2g.reshape(1, D), ln2b.reshape(1, D), ln3g.reshape(1, D),
        ln3b.reshape(1, D), W4, b4.reshape(1, 1))
    return out3.reshape(B, L)


# speedup vs baseline: 3.4532x; 1.3133x over previous
import math

import jax
import jax.numpy as jnp
from jax import lax
from jax.experimental import pallas as pl
from jax.experimental.pallas import tpu as pltpu
from jax.experimental.pallas import tpu_sc as plsc

B, L, D, H, V = 1024, 200, 256, 8, 100000
DH = D // H
NW = 32
CHUNK = 256


def _gather_rows_sc(table, idx):
    n = idx.shape[0]
    per_w = n // NW
    iters = per_w // CHUNK
    mesh = plsc.VectorSubcoreMesh(core_axis_name="c", subcore_axis_name="s")

    def body(table_ref, idx_ref, out_ref, idx_v, rows_v, sem):
        wid = lax.axis_index("s") * 2 + lax.axis_index("c")
        base = wid * per_w

        def step(i, carry):
            off = base + i * CHUNK
            pltpu.sync_copy(idx_ref.at[pl.ds(off, CHUNK)], idx_v)
            pltpu.async_copy(table_ref.at[idx_v], rows_v, sem).wait()
            pltpu.sync_copy(rows_v, out_ref.at[pl.ds(off, CHUNK)])
            return carry

        lax.fori_loop(0, iters, step, 0)

    fn = pl.kernel(
        body,
        out_type=jax.ShapeDtypeStruct((n, D), jnp.float32),
        mesh=mesh,
        scratch_types=[
            pltpu.VMEM((CHUNK,), jnp.int32),
            pltpu.VMEM((CHUNK, D), jnp.float32),
            pltpu.SemaphoreType.DMA,
        ],
    )
    return fn(table, idx)


def _ln_rows(x, g, b):
    m = jnp.mean(x, axis=-1, keepdims=True)
    xc = x - m
    v = jnp.mean(xc * xc, axis=-1, keepdims=True)
    return xc * lax.rsqrt(v + 1e-5) * g + b


def _transformer_body(he_ref, ne_ref, af_ref, P_ref, cd_ref, nb_ref,
                      W0a_ref, Wkf_ref, Wvf_ref, W3_ref, Wq_ref, bq_ref,
                      bk_ref, bv_ref, Wo_ref, bo_ref, F0w_ref, F1w_ref,
                      F0b_ref, F1b_ref, g2_ref, b2_ref, g3_ref, b3_ref,
                      W4_ref, b4_ref, out_ref):
    bf = jnp.bfloat16
    f32 = jnp.float32

    he = he_ref[0].astype(bf)
    ne = ne_ref[0].astype(bf)
    af = af_ref[0]
    nb = nb_ref[...]

    inter = (jnp.dot(he, W0a_ref[...], preferred_element_type=f32)
             + P_ref[...] + af * cd_ref[...]).astype(bf)
    k = (jnp.dot(inter, Wkf_ref[...], preferred_element_type=f32)
         + bk_ref[...]).astype(bf)
    v = (jnp.dot(inter, Wvf_ref[...], preferred_element_type=f32)
         + bv_ref[...]).astype(bf)
    query = jnp.dot(ne, W3_ref[...], preferred_element_type=f32)
    query16 = query.astype(bf)
    q = (jnp.dot(query16, Wq_ref[...], preferred_element_type=f32)
         + bq_ref[...]).astype(bf)

    ctxs = []
    for h in range(H):
        sl = slice(h * DH, (h + 1) * DH)
        s = lax.dot_general(q[:, sl], k[:, sl], (((1,), (1,)), ((), ())),
                            preferred_element_type=f32)
        e = jnp.exp(s + nb)
        r = 1.0 / jnp.sum(e, axis=1, keepdims=True)
        ctxs.append(
            jnp.dot(e.astype(bf), v[:, sl], preferred_element_type=f32) * r)
    ctx = jnp.concatenate(ctxs, axis=1).astype(bf)

    atn = (jnp.dot(ctx, Wo_ref[...], preferred_element_type=f32)
           + bo_ref[...] + query)
    atn = _ln_rows(atn, g2_ref[...], b2_ref[...])
    hdn = jnp.maximum(
        jnp.dot(atn.astype(bf), F0w_ref[...], preferred_element_type=f32)
        + F0b_ref[...], 0.0).astype(bf)
    ffn = jnp.dot(hdn, F1w_ref[...], preferred_element_type=f32) + F1b_ref[...]
    ffn = _ln_rows(ffn + atn, g3_ref[...], b3_ref[...])
    out_ref[0] = jnp.dot(ffn, W4_ref[...]) + b4_ref[...]


def _transformer_tc(he, ne, ansf, P, cdelta, nbias, W0a, Wkf, Wvf, W3, Wq,
                    bq, bk, bv, Wo, bo, F0w, F1w, F0b, F1b, g2, b2, g3, b3,
                    W4, b4):
    def blk(shape, imap):
        return pl.BlockSpec(shape, imap)

    row = lambda i: (i, 0, 0)
    const2 = lambda i: (0, 0)
    in_specs = [
        blk((1, L, D), row),
        blk((1, L, D), row),
        blk((1, L, 1), row),
        blk((L, D), const2),
        blk((1, D), const2),
        blk((L, L), const2),
        blk((D, D), const2),
        blk((D, D), const2),
        blk((D, D), const2),
        blk((D, D), const2),
        blk((D, D), const2),
        blk((1, D), const2),
        blk((1, D), const2),
        blk((1, D), const2),
        blk((D, D), const2),
        blk((1, D), const2),
        blk((D, D), const2),
        blk((D, D), const2),
        blk((1, D), const2),
        blk((1, D), const2),
        blk((1, D), const2),
        blk((1, D), const2),
        blk((1, D), const2),
        blk((1, D), const2),
        blk((D, 1), const2),
        blk((1, 1), const2),
    ]
    out = pl.pallas_call(
        _transformer_body,
        grid=(B,),
        in_specs=in_specs,
        out_specs=pl.BlockSpec((1, L, 1), row),
        out_shape=jax.ShapeDtypeStruct((B, L, 1), jnp.float32),
        compiler_params=pltpu.CompilerParams(
            dimension_semantics=("arbitrary",)),
    )(he, ne, ansf, P, cdelta, nbias, W0a, Wkf, Wvf, W3, Wq, bq, bk, bv, Wo,
      bo, F0w, F1w, F0b, F1b, g2, b2, g3, b3, W4, b4)
    return out


def kernel(hist_seq, hist_answers, new_seq, node_emb, corr_emb, pos_emb, W0,
           b0, W1, W2, W3, Wq, bq, Wk, bk, Wv, bv, Wo, bo, F0w, F0b, F1w,
           F1b, ln2g, ln2b, ln3g, ln3b, W4, b4):
    idx_all = jnp.concatenate(
        [hist_seq.reshape(-1), new_seq.reshape(-1)]).astype(jnp.int32)
    gathered = _gather_rows_sc(node_emb, idx_all)
    he = gathered[: B * L].reshape(B, L, D)
    ne = gathered[B * L:].reshape(B, L, D)

    bf = jnp.bfloat16
    corrW = corr_emb @ W0[D:]
    P = pos_emb + b0[None, :] + corrW[0][None, :]
    cdelta = (corrW[1] - corrW[0]).reshape(1, D)
    scale = jnp.float32(1.0 / math.sqrt(DH))
    cols_gt_rows = (jnp.arange(L)[None, :] > jnp.arange(L)[:, None])
    nbias = jnp.where(cols_gt_rows, jnp.float32(-1e30), 0.0)
    W0a = W0[:D].astype(bf)
    Wkf = (W2 @ Wk).astype(bf)
    Wvf = (W1 @ Wv).astype(bf)
    W3b = W3.astype(bf)
    Wqb = (Wq * scale).astype(bf)
    Wob = Wo.astype(bf)
    F0wb = F0w.astype(bf)
    F1wb = F1w.astype(bf)
    ansf = hist_answers.astype(jnp.float32).reshape(B, L, 1)

    out3 = _transformer_tc(
        he, ne, ansf, P, cdelta, nbias, W0a, Wkf, Wvf, W3b, Wqb,
        (bq * scale).reshape(1, D),
        bk.reshape(1, D), bv.reshape(1, D), Wob,
        bo.reshape(1, D), F0wb, F1wb,
        F0b.reshape(1, D), F1b.reshape(1, D),
        ln2g.reshape(1, D), ln2b.reshape(1, D), ln3g.reshape(1, D),
        ln3b.reshape(1, D), W4, b4.reshape(1, 1))
    return out3.reshape(B, L)


# speedup vs baseline: 5.5169x; 1.5976x over previous
import math

import jax
import jax.numpy as jnp
from jax import lax
from jax.experimental import pallas as pl
from jax.experimental.pallas import tpu as pltpu
from jax.experimental.pallas import tpu_sc as plsc

B, L, D, H, V = 1024, 200, 256, 8, 100000
DH = D // H
NW = 32
CHUNK = 256


def _gather_rows_sc(table, idx):
    n = idx.shape[0]
    per_w = n // NW
    iters = per_w // CHUNK
    mesh = plsc.VectorSubcoreMesh(core_axis_name="c", subcore_axis_name="s")

    def body(table_ref, idx_ref, out_ref, idx_v, rows_v, sem):
        wid = lax.axis_index("s") * 2 + lax.axis_index("c")
        base = wid * per_w

        def step(i, carry):
            off = base + i * CHUNK
            pltpu.sync_copy(idx_ref.at[pl.ds(off, CHUNK)], idx_v)
            pltpu.async_copy(table_ref.at[idx_v], rows_v, sem).wait()
            pltpu.sync_copy(rows_v, out_ref.at[pl.ds(off, CHUNK)])
            return carry

        lax.fori_loop(0, iters, step, 0)

    fn = pl.kernel(
        body,
        out_type=jax.ShapeDtypeStruct((n, D), jnp.float32),
        mesh=mesh,
        scratch_types=[
            pltpu.VMEM((CHUNK,), jnp.int32),
            pltpu.VMEM((CHUNK, D), jnp.float32),
            pltpu.SemaphoreType.DMA,
        ],
    )
    return fn(table, idx)


def _ln_rows(x, g, b):
    m = jnp.mean(x, axis=-1, keepdims=True)
    xc = x - m
    v = jnp.mean(xc * xc, axis=-1, keepdims=True)
    return xc * lax.rsqrt(v + 1e-5) * g + b


BB = 2


def _transformer_body(he_ref, ne_ref, af_ref, nb_ref, ones_ref, Pk_ref,
                      Pv_ref, cdk_ref, cdv_ref, Ak_ref, Av_ref, W3_ref,
                      Aq_ref, bq_ref, Wo_ref, bo_ref, F0w_ref, F1w_ref,
                      F0b_ref, F1b_ref, g2_ref, b2_ref, g3_ref, b3_ref,
                      W4_ref, b4_ref, out_ref):
    bf = jnp.bfloat16
    f32 = jnp.float32

    hef = he_ref[...].reshape(BB * L, D).astype(bf)
    nef = ne_ref[...].reshape(BB * L, D).astype(bf)
    af3 = af_ref[...]
    nb = nb_ref[...]
    ones_c = ones_ref[...]

    k3 = (jnp.dot(hef, Ak_ref[...],
                  preferred_element_type=f32).reshape(BB, L, D)
          + Pk_ref[...][None] + af3 * cdk_ref[...]).astype(bf)
    v3 = (jnp.dot(hef, Av_ref[...],
                  preferred_element_type=f32).reshape(BB, L, D)
          + Pv_ref[...][None] + af3 * cdv_ref[...]).astype(bf)
    query = jnp.dot(nef, W3_ref[...], preferred_element_type=f32)
    q3 = (jnp.dot(nef, Aq_ref[...],
                  preferred_element_type=f32).reshape(BB, L, D)
          + bq_ref[...]).astype(bf)

    ctx_rows = []
    for b in range(BB):
        q2, k2, v2 = q3[b], k3[b], v3[b]
        heads = []
        for h in range(H):
            sl = slice(h * DH, (h + 1) * DH)
            s = lax.dot_general(q2[:, sl], k2[:, sl],
                                (((1,), (1,)), ((), ())),
                                preferred_element_type=f32)
            e = jnp.exp(s + nb).astype(bf)
            vaug = jnp.concatenate([v2[:, sl], ones_c], axis=1)
            o = jnp.dot(e, vaug, preferred_element_type=f32)
            heads.append(o[:, :DH] * (1.0 / o[:, DH:DH + 1]))
        ctx_rows.append(jnp.concatenate(heads, axis=1))
    ctx = jnp.concatenate(ctx_rows, axis=0).astype(bf)

    atn = (jnp.dot(ctx, Wo_ref[...], preferred_element_type=f32)
           + bo_ref[...] + query)
    atn = _ln_rows(atn, g2_ref[...], b2_ref[...])
    hdn = jnp.maximum(
        jnp.dot(atn.astype(bf), F0w_ref[...], preferred_element_type=f32)
        + F0b_ref[...], 0.0).astype(bf)
    ffn = jnp.dot(hdn, F1w_ref[...], preferred_element_type=f32) + F1b_ref[...]
    ffn = _ln_rows(ffn + atn, g3_ref[...], b3_ref[...])
    pred = (jnp.dot(ffn.astype(bf), W4_ref[...], preferred_element_type=f32)
            + b4_ref[...])
    out_ref[...] = pred.reshape(BB, L, 1)


def _transformer_tc(he, ne, ansf, nbias, ones_c, Pk, Pv, cdk, cdv, Ak, Av,
                    W3, Aq, bq, Wo, bo, F0w, F1w, F0b, F1b, g2, b2, g3, b3,
                    W4, b4):
    def blk(shape, imap):
        return pl.BlockSpec(shape, imap)

    row = lambda i: (i, 0, 0)
    const2 = lambda i: (0, 0)
    in_specs = [
        blk((BB, L, D), row),
        blk((BB, L, D), row),
        blk((BB, L, 1), row),
        blk((L, L), const2),
        blk((L, 1), const2),
        blk((L, D), const2),
        blk((L, D), const2),
        blk((1, D), const2),
        blk((1, D), const2),
        blk((D, D), const2),
        blk((D, D), const2),
        blk((D, D), const2),
        blk((D, D), const2),
        blk((1, D), const2),
        blk((D, D), const2),
        blk((1, D), const2),
        blk((D, D), const2),
        blk((D, D), const2),
        blk((1, D), const2),
        blk((1, D), const2),
        blk((1, D), const2),
        blk((1, D), const2),
        blk((1, D), const2),
        blk((1, D), const2),
        blk((D, 1), const2),
        blk((1, 1), const2),
    ]
    out = pl.pallas_call(
        _transformer_body,
        grid=(B // BB,),
        in_specs=in_specs,
        out_specs=pl.BlockSpec((BB, L, 1), row),
        out_shape=jax.ShapeDtypeStruct((B, L, 1), jnp.float32),
        compiler_params=pltpu.CompilerParams(
            dimension_semantics=("arbitrary",)),
    )(he, ne, ansf, nbias, ones_c, Pk, Pv, cdk, cdv, Ak, Av, W3, Aq, bq, Wo,
      bo, F0w, F1w, F0b, F1b, g2, b2, g3, b3, W4, b4)
    return out


def kernel(hist_seq, hist_answers, new_seq, node_emb, corr_emb, pos_emb, W0,
           b0, W1, W2, W3, Wq, bq, Wk, bk, Wv, bv, Wo, bo, F0w, F0b, F1w,
           F1b, ln2g, ln2b, ln3g, ln3b, W4, b4):
    idx_all = jnp.concatenate(
        [hist_seq.reshape(-1), new_seq.reshape(-1)]).astype(jnp.int32)
    gathered = _gather_rows_sc(node_emb, idx_all)
    he = gathered[: B * L].reshape(B, L, D)
    ne = gathered[B * L:].reshape(B, L, D)

    bf = jnp.bfloat16
    corrW = corr_emb @ W0[D:]
    P = pos_emb + b0[None, :] + corrW[0][None, :]
    cdelta = (corrW[1] - corrW[0]).reshape(1, D)
    scale = jnp.float32(1.0 / math.sqrt(DH))
    cols_gt_rows = (jnp.arange(L)[None, :] > jnp.arange(L)[:, None])
    nbias = jnp.where(cols_gt_rows, jnp.float32(-1e30), 0.0)
    ones_c = jnp.ones((L, 1), bf)
    Wkf = W2 @ Wk
    Wvf = W1 @ Wv
    Ak = (W0[:D] @ Wkf).astype(bf)
    Av = (W0[:D] @ Wvf).astype(bf)
    Pk = P @ Wkf + bk[None, :]
    Pv = P @ Wvf + bv[None, :]
    cdk = cdelta @ Wkf
    cdv = cdelta @ Wvf
    W3b = W3.astype(bf)
    Aq = ((W3 @ Wq) * scale).astype(bf)
    Wob = Wo.astype(bf)
    F0wb = F0w.astype(bf)
    F1wb = F1w.astype(bf)
    W4b = W4.astype(bf)
    ansf = hist_answers.astype(jnp.float32).reshape(B, L, 1)

    out3 = _transformer_tc(
        he, ne, ansf, nbias, ones_c, Pk, Pv, cdk, cdv, Ak, Av, W3b, Aq,
        (bq * scale).reshape(1, D), Wob, bo.reshape(1, D), F0wb, F1wb,
        F0b.reshape(1, D), F1b.reshape(1, D),
        ln2g.reshape(1, D), ln2b.reshape(1, D), ln3g.reshape(1, D),
        ln3b.reshape(1, D), W4b, b4.reshape(1, 1))
    return out3.reshape(B, L)


# speedup vs baseline: 6.4771x; 1.1741x over previous
import math

import jax
import jax.numpy as jnp
from jax import lax
from jax.experimental import pallas as pl
from jax.experimental.pallas import tpu as pltpu
from jax.experimental.pallas import tpu_sc as plsc

B, L, D, H, V = 1024, 200, 256, 8, 100000
DH = D // H
NW = 32
CHUNK = 256


def _gather_rows_sc(table, idx):
    n = idx.shape[0]
    per_w = n // NW
    iters = per_w // CHUNK
    mesh = plsc.VectorSubcoreMesh(core_axis_name="c", subcore_axis_name="s")

    def body(table_ref, idx_ref, out_ref, idx_v, rows_v, sem):
        wid = lax.axis_index("s") * 2 + lax.axis_index("c")
        base = wid * per_w

        def step(i, carry):
            off = base + i * CHUNK
            pltpu.sync_copy(idx_ref.at[pl.ds(off, CHUNK)], idx_v)
            pltpu.async_copy(table_ref.at[idx_v], rows_v, sem).wait()
            pltpu.sync_copy(rows_v, out_ref.at[pl.ds(off, CHUNK)])
            return carry

        lax.fori_loop(0, iters, step, 0)

    fn = pl.kernel(
        body,
        out_type=jax.ShapeDtypeStruct((n, D), jnp.float32),
        mesh=mesh,
        scratch_types=[
            pltpu.VMEM((CHUNK,), jnp.int32),
            pltpu.VMEM((CHUNK, D), jnp.float32),
            pltpu.SemaphoreType.DMA,
        ],
    )
    return fn(table, idx)


def _ln_rows(x, g, b):
    m = jnp.mean(x, axis=-1, keepdims=True)
    xc = x - m
    v = jnp.mean(xc * xc, axis=-1, keepdims=True)
    return xc * lax.rsqrt(v + 1e-5) * g + b


BB = 4


def _transformer_body(he_ref, ne_ref, af_ref, nb_ref, ones_ref, Pk_ref,
                      Pv_ref, cdk_ref, cdv_ref, Ak_ref, Av_ref, W3_ref,
                      Aq_ref, bq_ref, Wo_ref, bo_ref, F0w_ref, F1w_ref,
                      F0b_ref, F1b_ref, g2_ref, b2_ref, g3_ref, b3_ref,
                      W4_ref, b4_ref, out_ref):
    bf = jnp.bfloat16
    f32 = jnp.float32

    hef = he_ref[...].reshape(BB * L, D).astype(bf)
    nef = ne_ref[...].reshape(BB * L, D).astype(bf)
    af3 = af_ref[...]
    nb = nb_ref[...]
    ones_c = ones_ref[...]

    k3 = (jnp.dot(hef, Ak_ref[...],
                  preferred_element_type=f32).reshape(BB, L, D)
          + Pk_ref[...][None] + af3 * cdk_ref[...]).astype(bf)
    v3 = (jnp.dot(hef, Av_ref[...],
                  preferred_element_type=f32).reshape(BB, L, D)
          + Pv_ref[...][None] + af3 * cdv_ref[...]).astype(bf)
    query = jnp.dot(nef, W3_ref[...], preferred_element_type=f32)
    q3 = (jnp.dot(nef, Aq_ref[...],
                  preferred_element_type=f32).reshape(BB, L, D)
          + bq_ref[...]).astype(bf)

    ctx_rows = []
    for b in range(BB):
        q2, k2, v2 = q3[b], k3[b], v3[b]
        heads = []
        for h in range(H):
            sl = slice(h * DH, (h + 1) * DH)
            s = lax.dot_general(q2[:, sl], k2[:, sl],
                                (((1,), (1,)), ((), ())),
                                preferred_element_type=f32)
            e = jnp.exp(s + nb).astype(bf)
            vaug = jnp.concatenate([v2[:, sl], ones_c], axis=1)
            o = jnp.dot(e, vaug, preferred_element_type=f32)
            heads.append(o[:, :DH] * (1.0 / o[:, DH:DH + 1]))
        ctx_rows.append(jnp.concatenate(heads, axis=1))
    ctx = jnp.concatenate(ctx_rows, axis=0).astype(bf)

    atn = (jnp.dot(ctx, Wo_ref[...], preferred_element_type=f32)
           + bo_ref[...] + query)
    atn = _ln_rows(atn, g2_ref[...], b2_ref[...])
    hdn = jnp.maximum(
        jnp.dot(atn.astype(bf), F0w_ref[...], preferred_element_type=f32)
        + F0b_ref[...], 0.0).astype(bf)
    ffn = jnp.dot(hdn, F1w_ref[...], preferred_element_type=f32) + F1b_ref[...]
    ffn = _ln_rows(ffn + atn, g3_ref[...], b3_ref[...])
    pred = (jnp.dot(ffn.astype(bf), W4_ref[...], preferred_element_type=f32)
            + b4_ref[...])
    out_ref[...] = pred.reshape(BB, L, 1)


def _transformer_tc(he, ne, ansf, nbias, ones_c, Pk, Pv, cdk, cdv, Ak, Av,
                    W3, Aq, bq, Wo, bo, F0w, F1w, F0b, F1b, g2, b2, g3, b3,
                    W4, b4):
    def blk(shape, imap):
        return pl.BlockSpec(shape, imap)

    row = lambda i: (i, 0, 0)
    const2 = lambda i: (0, 0)
    in_specs = [
        blk((BB, L, D), row),
        blk((BB, L, D), row),
        blk((BB, L, 1), row),
        blk((L, L), const2),
        blk((L, 1), const2),
        blk((L, D), const2),
        blk((L, D), const2),
        blk((1, D), const2),
        blk((1, D), const2),
        blk((D, D), const2),
        blk((D, D), const2),
        blk((D, D), const2),
        blk((D, D), const2),
        blk((1, D), const2),
        blk((D, D), const2),
        blk((1, D), const2),
        blk((D, D), const2),
        blk((D, D), const2),
        blk((1, D), const2),
        blk((1, D), const2),
        blk((1, D), const2),
        blk((1, D), const2),
        blk((1, D), const2),
        blk((1, D), const2),
        blk((D, 1), const2),
        blk((1, 1), const2),
    ]
    out = pl.pallas_call(
        _transformer_body,
        grid=(B // BB,),
        in_specs=in_specs,
        out_specs=pl.BlockSpec((BB, L, 1), row),
        out_shape=jax.ShapeDtypeStruct((B, L, 1), jnp.float32),
        compiler_params=pltpu.CompilerParams(
            dimension_semantics=("arbitrary",)),
    )(he, ne, ansf, nbias, ones_c, Pk, Pv, cdk, cdv, Ak, Av, W3, Aq, bq, Wo,
      bo, F0w, F1w, F0b, F1b, g2, b2, g3, b3, W4, b4)
    return out


def kernel(hist_seq, hist_answers, new_seq, node_emb, corr_emb, pos_emb, W0,
           b0, W1, W2, W3, Wq, bq, Wk, bk, Wv, bv, Wo, bo, F0w, F0b, F1w,
           F1b, ln2g, ln2b, ln3g, ln3b, W4, b4):
    idx_all = jnp.concatenate(
        [hist_seq.reshape(-1), new_seq.reshape(-1)]).astype(jnp.int32)
    gathered = _gather_rows_sc(node_emb, idx_all)
    he = gathered[: B * L].reshape(B, L, D)
    ne = gathered[B * L:].reshape(B, L, D)

    bf = jnp.bfloat16
    corrW = corr_emb @ W0[D:]
    P = pos_emb + b0[None, :] + corrW[0][None, :]
    cdelta = (corrW[1] - corrW[0]).reshape(1, D)
    scale = jnp.float32(1.0 / math.sqrt(DH))
    cols_gt_rows = (jnp.arange(L)[None, :] > jnp.arange(L)[:, None])
    nbias = jnp.where(cols_gt_rows, jnp.float32(-1e30), 0.0)
    ones_c = jnp.ones((L, 1), bf)
    Wkf = W2 @ Wk
    Wvf = W1 @ Wv
    Ak = (W0[:D] @ Wkf).astype(bf)
    Av = (W0[:D] @ Wvf).astype(bf)
    Pk = P @ Wkf + bk[None, :]
    Pv = P @ Wvf + bv[None, :]
    cdk = cdelta @ Wkf
    cdv = cdelta @ Wvf
    W3b = W3.astype(bf)
    Aq = ((W3 @ Wq) * scale).astype(bf)
    Wob = Wo.astype(bf)
    F0wb = F0w.astype(bf)
    F1wb = F1w.astype(bf)
    W4b = W4.astype(bf)
    ansf = hist_answers.astype(jnp.float32).reshape(B, L, 1)

    out3 = _transformer_tc(
        he, ne, ansf, nbias, ones_c, Pk, Pv, cdk, cdv, Ak, Av, W3b, Aq,
        (bq * scale).reshape(1, D), Wob, bo.reshape(1, D), F0wb, F1wb,
        F0b.reshape(1, D), F1b.reshape(1, D),
        ln2g.reshape(1, D), ln2b.reshape(1, D), ln3g.reshape(1, D),
        ln3b.reshape(1, D), W4b, b4.reshape(1, 1))
    return out3.reshape(B, L)


# speedup vs baseline: 6.7322x; 1.0394x over previous
import math

import jax
import jax.numpy as jnp
from jax import lax
from jax.experimental import pallas as pl
from jax.experimental.pallas import tpu as pltpu
from jax.experimental.pallas import tpu_sc as plsc

B, L, D, H, V = 1024, 200, 256, 8, 100000
DH = D // H
NW = 32
CHUNK = 256


def _gather_rows_sc(table, idx):
    n = idx.shape[0]
    per_w = n // NW
    iters = per_w // CHUNK
    mesh = plsc.VectorSubcoreMesh(core_axis_name="c", subcore_axis_name="s")

    def body(table_ref, idx_ref, out_ref, idx_v, rows_v, sem):
        wid = lax.axis_index("s") * 2 + lax.axis_index("c")
        base = wid * per_w

        def step(i, carry):
            off = base + i * CHUNK
            pltpu.sync_copy(idx_ref.at[pl.ds(off, CHUNK)], idx_v)
            pltpu.async_copy(table_ref.at[idx_v], rows_v, sem).wait()
            pltpu.sync_copy(rows_v, out_ref.at[pl.ds(off, CHUNK)])
            return carry

        lax.fori_loop(0, iters, step, 0)

    fn = pl.kernel(
        body,
        out_type=jax.ShapeDtypeStruct((n, D), jnp.float32),
        mesh=mesh,
        scratch_types=[
            pltpu.VMEM((CHUNK,), jnp.int32),
            pltpu.VMEM((CHUNK, D), jnp.float32),
            pltpu.SemaphoreType.DMA,
        ],
    )
    return fn(table, idx)


def _ln_rows(x, g, b):
    m = jnp.mean(x, axis=-1, keepdims=True)
    xc = x - m
    v = jnp.mean(xc * xc, axis=-1, keepdims=True)
    return xc * lax.rsqrt(v + 1e-5) * g + b


BB = 8


def _transformer_body(he_ref, ne_ref, af_ref, nb_ref, ones_ref, Pk_ref,
                      Pv_ref, cdk_ref, cdv_ref, Ak_ref, Av_ref, W3_ref,
                      Aq_ref, bq_ref, Wo_ref, bo_ref, F0w_ref, F1w_ref,
                      F0b_ref, F1b_ref, g2_ref, b2_ref, g3_ref, b3_ref,
                      W4_ref, b4_ref, out_ref):
    bf = jnp.bfloat16
    f32 = jnp.float32

    hef = he_ref[...].reshape(BB * L, D).astype(bf)
    nef = ne_ref[...].reshape(BB * L, D).astype(bf)
    af3 = af_ref[...]
    nb = nb_ref[...]
    ones_c = ones_ref[...]

    k3 = (jnp.dot(hef, Ak_ref[...],
                  preferred_element_type=f32).reshape(BB, L, D)
          + Pk_ref[...][None] + af3 * cdk_ref[...]).astype(bf)
    v3 = (jnp.dot(hef, Av_ref[...],
                  preferred_element_type=f32).reshape(BB, L, D)
          + Pv_ref[...][None] + af3 * cdv_ref[...]).astype(bf)
    query = jnp.dot(nef, W3_ref[...], preferred_element_type=f32)
    q3 = (jnp.dot(nef, Aq_ref[...],
                  preferred_element_type=f32).reshape(BB, L, D)
          + bq_ref[...]).astype(bf)

    ctx_rows = []
    for b in range(BB):
        q2, k2, v2 = q3[b], k3[b], v3[b]
        heads = []
        for h in range(H):
            sl = slice(h * DH, (h + 1) * DH)
            s = lax.dot_general(q2[:, sl], k2[:, sl],
                                (((1,), (1,)), ((), ())),
                                preferred_element_type=f32)
            e = jnp.exp(s + nb).astype(bf)
            vaug = jnp.concatenate([v2[:, sl], ones_c], axis=1)
            o = jnp.dot(e, vaug, preferred_element_type=f32)
            heads.append(o[:, :DH] * (1.0 / o[:, DH:DH + 1]))
        ctx_rows.append(jnp.concatenate(heads, axis=1))
    ctx = jnp.concatenate(ctx_rows, axis=0).astype(bf)

    atn = (jnp.dot(ctx, Wo_ref[...], preferred_element_type=f32)
           + bo_ref[...] + query)
    atn = _ln_rows(atn, g2_ref[...], b2_ref[...])
    hdn = jnp.maximum(
        jnp.dot(atn.astype(bf), F0w_ref[...], preferred_element_type=f32)
        + F0b_ref[...], 0.0).astype(bf)
    ffn = jnp.dot(hdn, F1w_ref[...], preferred_element_type=f32) + F1b_ref[...]
    ffn = _ln_rows(ffn + atn, g3_ref[...], b3_ref[...])
    pred = (jnp.dot(ffn.astype(bf), W4_ref[...], preferred_element_type=f32)
            + b4_ref[...])
    out_ref[...] = pred.reshape(BB, L, 1)


def _transformer_tc(he, ne, ansf, nbias, ones_c, Pk, Pv, cdk, cdv, Ak, Av,
                    W3, Aq, bq, Wo, bo, F0w, F1w, F0b, F1b, g2, b2, g3, b3,
                    W4, b4):
    def blk(shape, imap):
        return pl.BlockSpec(shape, imap)

    row = lambda i: (i, 0, 0)
    const2 = lambda i: (0, 0)
    in_specs = [
        blk((BB, L, D), row),
        blk((BB, L, D), row),
        blk((BB, L, 1), row),
        blk((L, L), const2),
        blk((L, 1), const2),
        blk((L, D), const2),
        blk((L, D), const2),
        blk((1, D), const2),
        blk((1, D), const2),
        blk((D, D), const2),
        blk((D, D), const2),
        blk((D, D), const2),
        blk((D, D), const2),
        blk((1, D), const2),
        blk((D, D), const2),
        blk((1, D), const2),
        blk((D, D), const2),
        blk((D, D), const2),
        blk((1, D), const2),
        blk((1, D), const2),
        blk((1, D), const2),
        blk((1, D), const2),
        blk((1, D), const2),
        blk((1, D), const2),
        blk((D, 1), const2),
        blk((1, 1), const2),
    ]
    out = pl.pallas_call(
        _transformer_body,
        grid=(B // BB,),
        in_specs=in_specs,
        out_specs=pl.BlockSpec((BB, L, 1), row),
        out_shape=jax.ShapeDtypeStruct((B, L, 1), jnp.float32),
        compiler_params=pltpu.CompilerParams(
            dimension_semantics=("arbitrary",)),
    )(he, ne, ansf, nbias, ones_c, Pk, Pv, cdk, cdv, Ak, Av, W3, Aq, bq, Wo,
      bo, F0w, F1w, F0b, F1b, g2, b2, g3, b3, W4, b4)
    return out


def kernel(hist_seq, hist_answers, new_seq, node_emb, corr_emb, pos_emb, W0,
           b0, W1, W2, W3, Wq, bq, Wk, bk, Wv, bv, Wo, bo, F0w, F0b, F1w,
           F1b, ln2g, ln2b, ln3g, ln3b, W4, b4):
    idx_all = jnp.concatenate(
        [hist_seq.reshape(-1), new_seq.reshape(-1)]).astype(jnp.int32)
    gathered = _gather_rows_sc(node_emb, idx_all)
    he = gathered[: B * L].reshape(B, L, D)
    ne = gathered[B * L:].reshape(B, L, D)

    bf = jnp.bfloat16
    corrW = corr_emb @ W0[D:]
    P = pos_emb + b0[None, :] + corrW[0][None, :]
    cdelta = (corrW[1] - corrW[0]).reshape(1, D)
    scale = jnp.float32(1.0 / math.sqrt(DH))
    cols_gt_rows = (jnp.arange(L)[None, :] > jnp.arange(L)[:, None])
    nbias = jnp.where(cols_gt_rows, jnp.float32(-1e30), 0.0)
    ones_c = jnp.ones((L, 1), bf)
    Wkf = W2 @ Wk
    Wvf = W1 @ Wv
    Ak = (W0[:D] @ Wkf).astype(bf)
    Av = (W0[:D] @ Wvf).astype(bf)
    Pk = P @ Wkf + bk[None, :]
    Pv = P @ Wvf + bv[None, :]
    cdk = cdelta @ Wkf
    cdv = cdelta @ Wvf
    W3b = W3.astype(bf)
    Aq = ((W3 @ Wq) * scale).astype(bf)
    Wob = Wo.astype(bf)
    F0wb = F0w.astype(bf)
    F1wb = F1w.astype(bf)
    W4b = W4.astype(bf)
    ansf = hist_answers.astype(jnp.float32).reshape(B, L, 1)

    out3 = _transformer_tc(
        he, ne, ansf, nbias, ones_c, Pk, Pv, cdk, cdv, Ak, Av, W3b, Aq,
        (bq * scale).reshape(1, D), Wob, bo.reshape(1, D), F0wb, F1wb,
        F0b.reshape(1, D), F1b.reshape(1, D),
        ln2g.reshape(1, D), ln2b.reshape(1, D), ln3g.reshape(1, D),
        ln3b.reshape(1, D), W4b, b4.reshape(1, 1))
    return out3.reshape(B, L)
